# baseline (speedup 1.0000x reference)
import functools

import jax
import jax.numpy as jnp
from jax import lax
from jax.experimental import pallas as pl
from jax.experimental.pallas import tpu as pltpu
from jax.experimental.pallas import tpu_sc as plsc

B, K, NE, NH = 16384, 26, 32, 128
NC, NS, L = 2, 16, 16
NW = NC * NS
ROWS_W = B // NW
CHUNK = 4
NIDX = CHUNK * K
NSTEP = ROWS_W // CHUNK
NBUF = 4
NOUTER = NSTEP // NBUF


def _sc_gather_mul(cnt_table, val_table, goal_flat):
    mesh = plsc.VectorSubcoreMesh(core_axis_name="c", subcore_axis_name="s")

    @functools.partial(
        pl.kernel,
        mesh=mesh,
        out_type=jax.ShapeDtypeStruct((B * K, NE), jnp.float32),
        scratch_types=[
            pltpu.VMEM((ROWS_W * 2 * K,), jnp.int32),
            pltpu.VMEM((ROWS_W * K,), jnp.int32),
            pltpu.VMEM((ROWS_W * K,), jnp.int32),
            pltpu.VMEM((NBUF, NIDX, NE), jnp.float32),
            pltpu.VMEM((NBUF, NIDX, NE), jnp.float32),
            pltpu.VMEM((NBUF, NIDX, NE), jnp.float32),
            [pltpu.SemaphoreType.DMA for _ in range(NBUF)],
            [pltpu.SemaphoreType.DMA for _ in range(NBUF)],
            [pltpu.SemaphoreType.DMA for _ in range(NBUF)],
        ],
        compiler_params=pltpu.CompilerParams(
            use_tc_tiling_on_sc=False, needs_layout_passes=False),
    )
    def k(cnt_hbm, val_hbm, goal_hbm, out_hbm,
          goal_v, cidx_v, vidx_v, crow_v, vrow_v, hrow_v, csems, vsems, osems):
        wid = lax.axis_index("s") * NC + lax.axis_index("c")
        ibase = wid * (ROWS_W * K)
        pltpu.sync_copy(goal_hbm.at[pl.ds(2 * ibase, ROWS_W * 2 * K)], goal_v)

        def deint(i, _):
            lane = lax.iota(jnp.int32, 16)
            t = i * 16
            cpos = 2 * (t + lane)
            cidx_v[pl.ds(t, 16)] = plsc.load_gather(goal_v, [cpos])
            vidx_v[pl.ds(t, 16)] = plsc.load_gather(goal_v, [cpos + 1])
            return 0

        lax.fori_loop(0, ROWS_W * K // 16, deint, 0)

        def start_gather(b, off):
            pltpu.async_copy(
                cnt_hbm.at[cidx_v.at[pl.ds(off, NIDX)]], crow_v.at[b],
                csems[b])
            pltpu.async_copy(
                val_hbm.at[vidx_v.at[pl.ds(off, NIDX)]], vrow_v.at[b],
                vsems[b])

        def wait_gather(b, off):
            pltpu.make_async_copy(
                cnt_hbm.at[cidx_v.at[pl.ds(off, NIDX)]], crow_v.at[b],
                csems[b]).wait()
            pltpu.make_async_copy(
                val_hbm.at[vidx_v.at[pl.ds(off, NIDX)]], vrow_v.at[b],
                vsems[b]).wait()

        for b in range(NBUF):
            start_gather(b, b * NIDX)

        def outer(o, _):
            for b in range(NBUF):
                off = (o * NBUF + b) * NIDX
                wait_gather(b, off)

                @pl.when(o >= 1)
                def _():
                    pltpu.make_async_copy(
                        hrow_v.at[b],
                        out_hbm.at[pl.ds(ibase + off - NBUF * NIDX, NIDX)],
                        osems[b]).wait()

                def mul(j, _):
                    for e in range(0, NE, L):
                        hrow_v[b, j, pl.ds(e, L)] = (
                            crow_v[b, j, pl.ds(e, L)]
                            * vrow_v[b, j, pl.ds(e, L)])
                    return 0

                lax.fori_loop(0, NIDX, mul, 0, unroll=4)
                pltpu.async_copy(
                    hrow_v.at[b], out_hbm.at[pl.ds(ibase + off, NIDX)],
                    osems[b])

                @pl.when(o < NOUTER - 1)
                def _():
                    start_gather(b, off + NBUF * NIDX)
            return 0

        lax.fori_loop(0, NOUTER, outer, 0)
        for b in range(NBUF):
            off = (NSTEP - NBUF + b) * NIDX
            pltpu.make_async_copy(
                hrow_v.at[b], out_hbm.at[pl.ds(ibase + off, NIDX)],
                osems[b]).wait()

    return k(cnt_table, val_table, goal_flat)


def _tc_body(h_ref, w_ref, b_ref, o_ref):
    h = jnp.tanh(h_ref[...])
    o_ref[...] = (
        jnp.dot(h, w_ref[...], preferred_element_type=jnp.float32)
        + b_ref[...])


def _tc_tanh_matmul(h, W, b):
    BM = 1024
    return pl.pallas_call(
        _tc_body,
        grid=(B // BM,),
        in_specs=[
            pl.BlockSpec((BM, K * NE), lambda i: (i, 0)),
            pl.BlockSpec((K * NE, NH), lambda i: (0, 0)),
            pl.BlockSpec((1, NH), lambda i: (0, 0)),
        ],
        out_specs=pl.BlockSpec((BM, NH), lambda i: (i, 0)),
        out_shape=jax.ShapeDtypeStruct((B, NH), jnp.float32),
    )(h, W, b)


@jax.jit
def kernel(goal, cnt_table, val_table, W, b):
    h = _sc_gather_mul(cnt_table, val_table, goal.reshape(-1))
    h = h.reshape(B, K * NE)
    return _tc_tanh_matmul(h, W, b.reshape(1, NH))


# speedup vs baseline: 1.0244x; 1.0244x over previous
import functools

import jax
import jax.numpy as jnp
from jax import lax
from jax.experimental import pallas as pl
from jax.experimental.pallas import tpu as pltpu
from jax.experimental.pallas import tpu_sc as plsc

B, K, NE, NH = 16384, 26, 32, 128
NC, NS, L = 2, 16, 16
NW = NC * NS
ROWS_W = B // NW
CHUNK = 4
NIDX = CHUNK * K
NSTEP = ROWS_W // CHUNK
NBUF = 4
NOUTER = NSTEP // NBUF


def _sc_gather_mul(cnt_table, val_table, goal_flat):
    mesh = plsc.VectorSubcoreMesh(core_axis_name="c", subcore_axis_name="s")

    @functools.partial(
        pl.kernel,
        mesh=mesh,
        out_type=jax.ShapeDtypeStruct((B, K * NE), jnp.float32),
        scratch_types=[
            pltpu.VMEM((ROWS_W * 2 * K,), jnp.int32),
            pltpu.VMEM((ROWS_W * K,), jnp.int32),
            pltpu.VMEM((ROWS_W * K,), jnp.int32),
            pltpu.VMEM((NBUF, NIDX, NE), jnp.float32),
            pltpu.VMEM((NBUF, NIDX, NE), jnp.float32),
            pltpu.VMEM((NBUF, CHUNK, K * NE), jnp.float32),
            [pltpu.SemaphoreType.DMA for _ in range(NBUF)],
            [pltpu.SemaphoreType.DMA for _ in range(NBUF)],
            [pltpu.SemaphoreType.DMA for _ in range(NBUF)],
        ],
        compiler_params=pltpu.CompilerParams(
            use_tc_tiling_on_sc=False, needs_layout_passes=False),
    )
    def k(cnt_hbm, val_hbm, goal_hbm, out_hbm,
          goal_v, cidx_v, vidx_v, crow_v, vrow_v, hrow_v, csems, vsems, osems):
        wid = lax.axis_index("s") * NC + lax.axis_index("c")
        wbase = wid * ROWS_W
        ibase = wid * (ROWS_W * K)
        pltpu.sync_copy(goal_hbm.at[pl.ds(2 * ibase, ROWS_W * 2 * K)], goal_v)

        def deint(i, _):
            lane = lax.iota(jnp.int32, 16)
            t = i * 16
            cpos = 2 * (t + lane)
            cidx_v[pl.ds(t, 16)] = plsc.load_gather(goal_v, [cpos])
            vidx_v[pl.ds(t, 16)] = plsc.load_gather(goal_v, [cpos + 1])
            return 0

        lax.fori_loop(0, ROWS_W * K // 16, deint, 0)

        def start_gather(b, off):
            pltpu.async_copy(
                cnt_hbm.at[cidx_v.at[pl.ds(off, NIDX)]], crow_v.at[b],
                csems[b])
            pltpu.async_copy(
                val_hbm.at[vidx_v.at[pl.ds(off, NIDX)]], vrow_v.at[b],
                vsems[b])

        def wait_gather(b, off):
            pltpu.make_async_copy(
                cnt_hbm.at[cidx_v.at[pl.ds(off, NIDX)]], crow_v.at[b],
                csems[b]).wait()
            pltpu.make_async_copy(
                val_hbm.at[vidx_v.at[pl.ds(off, NIDX)]], vrow_v.at[b],
                vsems[b]).wait()

        for b in range(NBUF):
            start_gather(b, b * NIDX)

        def outer(o, _):
            for b in range(NBUF):
                off = (o * NBUF + b) * NIDX
                wait_gather(b, off)

                row0 = wbase + (o * NBUF + b) * CHUNK

                @pl.when(o >= 1)
                def _():
                    pltpu.make_async_copy(
                        hrow_v.at[b],
                        out_hbm.at[pl.ds(row0 - NBUF * CHUNK, CHUNK)],
                        osems[b]).wait()

                for c in range(CHUNK):
                    def mul(kk, _):
                        for e in range(0, NE, L):
                            hrow_v[b, c, pl.ds(kk * NE + e, L)] = (
                                crow_v[b, c * K + kk, pl.ds(e, L)]
                                * vrow_v[b, c * K + kk, pl.ds(e, L)])
                        return 0

                    lax.fori_loop(0, K, mul, 0, unroll=4)
                pltpu.async_copy(
                    hrow_v.at[b], out_hbm.at[pl.ds(row0, CHUNK)],
                    osems[b])

                @pl.when(o < NOUTER - 1)
                def _():
                    start_gather(b, off + NBUF * NIDX)
            return 0

        lax.fori_loop(0, NOUTER, outer, 0)
        for b in range(NBUF):
            row0 = wbase + (NSTEP - NBUF + b) * CHUNK
            pltpu.make_async_copy(
                hrow_v.at[b], out_hbm.at[pl.ds(row0, CHUNK)],
                osems[b]).wait()

    return k(cnt_table, val_table, goal_flat)


def _tc_body(h_ref, w_ref, b_ref, o_ref):
    h = jnp.tanh(h_ref[...])
    o_ref[...] = (
        jnp.dot(h, w_ref[...], preferred_element_type=jnp.float32)
        + b_ref[...])


def _tc_tanh_matmul(h, W, b):
    BM = 1024
    return pl.pallas_call(
        _tc_body,
        grid=(B // BM,),
        in_specs=[
            pl.BlockSpec((BM, K * NE), lambda i: (i, 0)),
            pl.BlockSpec((K * NE, NH), lambda i: (0, 0)),
            pl.BlockSpec((1, NH), lambda i: (0, 0)),
        ],
        out_specs=pl.BlockSpec((BM, NH), lambda i: (i, 0)),
        out_shape=jax.ShapeDtypeStruct((B, NH), jnp.float32),
    )(h, W, b)


@jax.jit
def kernel(goal, cnt_table, val_table, W, b):
    h = _sc_gather_mul(cnt_table, val_table, goal.reshape(-1))
    return _tc_tanh_matmul(h, W, b.reshape(1, NH))


# speedup vs baseline: 1.7968x; 1.7539x over previous
import functools

import jax
import jax.numpy as jnp
from jax import lax
from jax.experimental import pallas as pl
from jax.experimental.pallas import tpu as pltpu
from jax.experimental.pallas import tpu_sc as plsc

B, K, NE, NH = 16384, 26, 32, 128
NC, NS, L = 2, 16, 16
NW = NC * NS
ROWS_W = B // NW
CHUNK = 4
NIDX = CHUNK * K
NSTEP = ROWS_W // CHUNK
NBUF = 4
NOUTER = NSTEP // NBUF


def _sc_gather_mul(cnt_table, val_table, goal_flat):
    mesh = plsc.VectorSubcoreMesh(core_axis_name="c", subcore_axis_name="s")

    @functools.partial(
        pl.kernel,
        mesh=mesh,
        out_type=jax.ShapeDtypeStruct((B, K * NE), jnp.float32),
        scratch_types=[
            pltpu.VMEM((ROWS_W * 2 * K,), jnp.int32),
            pltpu.VMEM((ROWS_W * K,), jnp.int32),
            pltpu.VMEM((ROWS_W * K,), jnp.int32),
            pltpu.VMEM((NBUF, NIDX, NE), jnp.float32),
            pltpu.VMEM((NBUF, NIDX, NE), jnp.float32),
            pltpu.VMEM((NBUF, CHUNK, K * NE), jnp.float32),
            [pltpu.SemaphoreType.DMA for _ in range(NBUF)],
            [pltpu.SemaphoreType.DMA for _ in range(NBUF)],
            [pltpu.SemaphoreType.DMA for _ in range(NBUF)],
        ],
        compiler_params=pltpu.CompilerParams(
            use_tc_tiling_on_sc=False, needs_layout_passes=False),
    )
    def k(cnt_hbm, val_hbm, goal_hbm, out_hbm,
          goal_v, cidx_v, vidx_v, crow_v, vrow_v, hrow_v, csems, vsems, osems):
        wid = lax.axis_index("s") * NC + lax.axis_index("c")
        wbase = wid * ROWS_W
        ibase = wid * (ROWS_W * K)
        pltpu.sync_copy(goal_hbm.at[pl.ds(2 * ibase, ROWS_W * 2 * K)], goal_v)

        def perm(i):
            return ((i & ~jnp.int32(511)) | ((i & 127) << 2)
                    | ((i >> 7) & 3))

        def deint(i, _):
            lane = lax.iota(jnp.int32, 16)
            t = i * 16
            cpos = 2 * (t + lane)
            cidx_v[pl.ds(t, 16)] = perm(plsc.load_gather(goal_v, [cpos]))
            vidx_v[pl.ds(t, 16)] = perm(plsc.load_gather(goal_v, [cpos + 1]))
            return 0

        lax.fori_loop(0, ROWS_W * K // 16, deint, 0)

        def start_gather(b, off):
            pltpu.async_copy(
                cnt_hbm.at[cidx_v.at[pl.ds(off, NIDX)]], crow_v.at[b],
                csems[b])
            pltpu.async_copy(
                val_hbm.at[vidx_v.at[pl.ds(off, NIDX)]], vrow_v.at[b],
                vsems[b])

        def wait_gather(b, off):
            pltpu.make_async_copy(
                cnt_hbm.at[cidx_v.at[pl.ds(off, NIDX)]], crow_v.at[b],
                csems[b]).wait()
            pltpu.make_async_copy(
                val_hbm.at[vidx_v.at[pl.ds(off, NIDX)]], vrow_v.at[b],
                vsems[b]).wait()

        for b in range(NBUF):
            start_gather(b, b * NIDX)

        def outer(o, _):
            for b in range(NBUF):
                off = (o * NBUF + b) * NIDX
                wait_gather(b, off)

                row0 = wbase + (o * NBUF + b) * CHUNK

                @pl.when(o >= 1)
                def _():
                    pltpu.make_async_copy(
                        hrow_v.at[b],
                        out_hbm.at[pl.ds(row0 - NBUF * CHUNK, CHUNK)],
                        osems[b]).wait()

                for c in range(CHUNK):
                    def mul(kk, _):
                        for e in range(0, NE, L):
                            hrow_v[b, c, pl.ds(kk * NE + e, L)] = (
                                crow_v[b, c * K + kk, pl.ds(e, L)]
                                * vrow_v[b, c * K + kk, pl.ds(e, L)])
                        return 0

                    lax.fori_loop(0, K, mul, 0, unroll=4)
                pltpu.async_copy(
                    hrow_v.at[b], out_hbm.at[pl.ds(row0, CHUNK)],
                    osems[b])

                @pl.when(o < NOUTER - 1)
                def _():
                    start_gather(b, off + NBUF * NIDX)
            return 0

        lax.fori_loop(0, NOUTER, outer, 0)
        for b in range(NBUF):
            row0 = wbase + (NSTEP - NBUF + b) * CHUNK
            pltpu.make_async_copy(
                hrow_v.at[b], out_hbm.at[pl.ds(row0, CHUNK)],
                osems[b]).wait()

    return k(cnt_table, val_table, goal_flat)


J4 = 4
TBN = 512 * J4


def _pack(x):
    z = x.reshape(32, J4, 4, 128)
    m = z.transpose(1, 2, 0, 3).reshape(J4, 128, 128)
    return jnp.swapaxes(m, 1, 2).reshape(TBN // 4, 128)


def _relayout_body(c_ref, v_ref, co_ref, vo_ref):
    co_ref[...] = _pack(c_ref[...])
    vo_ref[...] = _pack(v_ref[...])


def _tc_relayout(cnt_t, val_t):
    nrow = cnt_t.shape[1]
    g = (nrow + TBN - 1) // TBN
    return pl.pallas_call(
        _relayout_body,
        grid=(g,),
        in_specs=[
            pl.BlockSpec((32, TBN), lambda i: (0, i)),
            pl.BlockSpec((32, TBN), lambda i: (0, i)),
        ],
        out_specs=[
            pl.BlockSpec((TBN // 4, 128), lambda i: (i, 0)),
            pl.BlockSpec((TBN // 4, 128), lambda i: (i, 0)),
        ],
        out_shape=[
            jax.ShapeDtypeStruct((g * TBN // 4, 128), jnp.float32),
            jax.ShapeDtypeStruct((g * TBN // 4, 128), jnp.float32),
        ],
    )(cnt_t, val_t)


def _tc_body(h_ref, w_ref, b_ref, o_ref):
    h = jnp.tanh(h_ref[...])
    o_ref[...] = (
        jnp.dot(h, w_ref[...], preferred_element_type=jnp.float32)
        + b_ref[...])


def _tc_tanh_matmul(h, W, b):
    BM = 1024
    return pl.pallas_call(
        _tc_body,
        grid=(B // BM,),
        in_specs=[
            pl.BlockSpec((BM, K * NE), lambda i: (i, 0)),
            pl.BlockSpec((K * NE, NH), lambda i: (0, 0)),
            pl.BlockSpec((1, NH), lambda i: (0, 0)),
        ],
        out_specs=pl.BlockSpec((BM, NH), lambda i: (i, 0)),
        out_shape=jax.ShapeDtypeStruct((B, NH), jnp.float32),
    )(h, W, b)


@jax.jit
def kernel(goal, cnt_table, val_table, W, b):
    cnt_l, val_l = _tc_relayout(cnt_table.T, val_table.T)
    h = _sc_gather_mul(cnt_l.reshape(-1, NE), val_l.reshape(-1, NE),
                       goal.reshape(-1))
    return _tc_tanh_matmul(h, W, b.reshape(1, NH))


# speedup vs baseline: 2.5608x; 1.4252x over previous
import functools

import jax
import jax.numpy as jnp
from jax import lax
from jax.experimental import pallas as pl
from jax.experimental.pallas import tpu as pltpu
from jax.experimental.pallas import tpu_sc as plsc

B, K, NE, NH = 16384, 26, 32, 128
NC, NS, L = 2, 16, 16
NW = NC * NS
ROWS_W = B // NW
CHUNK = 4
NIDX = CHUNK * K
NSTEP = ROWS_W // CHUNK
NBUF = 4
NOUTER = NSTEP // NBUF


def _sc_gather_mul(cnt_table, val_table, goal_flat):
    mesh = plsc.VectorSubcoreMesh(core_axis_name="c", subcore_axis_name="s")

    @functools.partial(
        pl.kernel,
        mesh=mesh,
        out_type=jax.ShapeDtypeStruct((B, K * NE), jnp.float32),
        scratch_types=[
            pltpu.VMEM((ROWS_W * 2 * K,), jnp.int32),
            pltpu.VMEM((ROWS_W * K,), jnp.int32),
            pltpu.VMEM((ROWS_W * K,), jnp.int32),
            pltpu.VMEM((NBUF, NIDX, NE), jnp.float32),
            pltpu.VMEM((NBUF, NIDX, NE), jnp.float32),
            pltpu.VMEM((NBUF, CHUNK, K * NE), jnp.float32),
            [pltpu.SemaphoreType.DMA for _ in range(NBUF)],
            [pltpu.SemaphoreType.DMA for _ in range(NBUF)],
            [pltpu.SemaphoreType.DMA for _ in range(NBUF)],
        ],
        compiler_params=pltpu.CompilerParams(
            use_tc_tiling_on_sc=False, needs_layout_passes=False),
    )
    def k(cnt_hbm, val_hbm, goal_hbm, out_hbm,
          goal_v, cidx_v, vidx_v, crow_v, vrow_v, hrow_v, csems, vsems, osems):
        wid = lax.axis_index("s") * NC + lax.axis_index("c")
        wbase = wid * ROWS_W
        ibase = wid * (ROWS_W * K)
        pltpu.sync_copy(goal_hbm.at[pl.ds(2 * ibase, ROWS_W * 2 * K)], goal_v)

        def perm(i):
            return ((i & ~jnp.int32(511)) | ((i & 127) << 2)
                    | ((i >> 7) & 3))

        def deint(i, _):
            lane = lax.iota(jnp.int32, 16)
            t = i * 16
            cpos = 2 * (t + lane)
            cidx_v[pl.ds(t, 16)] = perm(plsc.load_gather(goal_v, [cpos]))
            vidx_v[pl.ds(t, 16)] = perm(plsc.load_gather(goal_v, [cpos + 1]))
            return 0

        lax.fori_loop(0, ROWS_W * K // 16, deint, 0)

        def start_gather(b, off):
            pltpu.async_copy(
                cnt_hbm.at[cidx_v.at[pl.ds(off, NIDX)]], crow_v.at[b],
                csems[b])
            pltpu.async_copy(
                val_hbm.at[vidx_v.at[pl.ds(off, NIDX)]], vrow_v.at[b],
                vsems[b])

        def wait_gather(b, off):
            pltpu.make_async_copy(
                cnt_hbm.at[cidx_v.at[pl.ds(off, NIDX)]], crow_v.at[b],
                csems[b]).wait()
            pltpu.make_async_copy(
                val_hbm.at[vidx_v.at[pl.ds(off, NIDX)]], vrow_v.at[b],
                vsems[b]).wait()

        for b in range(NBUF):
            start_gather(b, b * NIDX)

        def outer(o, _):
            for b in range(NBUF):
                off = (o * NBUF + b) * NIDX
                wait_gather(b, off)

                row0 = wbase + (o * NBUF + b) * CHUNK

                @pl.when(o >= 1)
                def _():
                    pltpu.make_async_copy(
                        hrow_v.at[b],
                        out_hbm.at[pl.ds(row0 - NBUF * CHUNK, CHUNK)],
                        osems[b]).wait()

                for c in range(CHUNK):
                    def mul(kk, _):
                        for e in range(0, NE, L):
                            hrow_v[b, c, pl.ds(kk * NE + e, L)] = (
                                crow_v[b, c * K + kk, pl.ds(e, L)]
                                * vrow_v[b, c * K + kk, pl.ds(e, L)])
                        return 0

                    lax.fori_loop(0, K, mul, 0, unroll=4)
                pltpu.async_copy(
                    hrow_v.at[b], out_hbm.at[pl.ds(row0, CHUNK)],
                    osems[b])

                @pl.when(o < NOUTER - 1)
                def _():
                    start_gather(b, off + NBUF * NIDX)
            return 0

        lax.fori_loop(0, NOUTER, outer, 0)
        for b in range(NBUF):
            row0 = wbase + (NSTEP - NBUF + b) * CHUNK
            pltpu.make_async_copy(
                hrow_v.at[b], out_hbm.at[pl.ds(row0, CHUNK)],
                osems[b]).wait()

    return k(cnt_table, val_table, goal_flat)


J4 = 16
TBN = 512 * J4


def _pack(x):
    z = x.reshape(32, J4, 4, 128)
    m = z.transpose(1, 2, 0, 3).reshape(J4, 128, 128)
    return jnp.swapaxes(m, 1, 2).reshape(TBN // 4, 128)


def _relayout_body(c_ref, v_ref, co_ref, vo_ref):
    co_ref[...] = _pack(c_ref[...])
    vo_ref[...] = _pack(v_ref[...])


def _tc_relayout(cnt_t, val_t):
    nrow = cnt_t.shape[1]
    g = (nrow + TBN - 1) // TBN
    return pl.pallas_call(
        _relayout_body,
        grid=(g,),
        in_specs=[
            pl.BlockSpec((32, TBN), lambda i: (0, i)),
            pl.BlockSpec((32, TBN), lambda i: (0, i)),
        ],
        out_specs=[
            pl.BlockSpec((TBN // 4, 128), lambda i: (i, 0)),
            pl.BlockSpec((TBN // 4, 128), lambda i: (i, 0)),
        ],
        out_shape=[
            jax.ShapeDtypeStruct((g * TBN // 4, 128), jnp.float32),
            jax.ShapeDtypeStruct((g * TBN // 4, 128), jnp.float32),
        ],
    )(cnt_t, val_t)


def _tc_body(h_ref, w_ref, b_ref, o_ref):
    h = jnp.tanh(h_ref[...])
    o_ref[...] = (
        jnp.dot(h, w_ref[...], preferred_element_type=jnp.float32)
        + b_ref[...])


def _tc_tanh_matmul(h, W, b):
    BM = 1024
    return pl.pallas_call(
        _tc_body,
        grid=(B // BM,),
        in_specs=[
            pl.BlockSpec((BM, K * NE), lambda i: (i, 0)),
            pl.BlockSpec((K * NE, NH), lambda i: (0, 0)),
            pl.BlockSpec((1, NH), lambda i: (0, 0)),
        ],
        out_specs=pl.BlockSpec((BM, NH), lambda i: (i, 0)),
        out_shape=jax.ShapeDtypeStruct((B, NH), jnp.float32),
    )(h, W, b)


@jax.jit
def kernel(goal, cnt_table, val_table, W, b):
    cnt_l, val_l = _tc_relayout(cnt_table.T, val_table.T)
    h = _sc_gather_mul(cnt_l.reshape(-1, NE), val_l.reshape(-1, NE),
                       goal.reshape(-1))
    return _tc_tanh_matmul(h, W, b.reshape(1, NH))


# speedup vs baseline: 2.7405x; 1.0701x over previous
import functools

import jax
import jax.numpy as jnp
from jax import lax
from jax.experimental import pallas as pl
from jax.experimental.pallas import tpu as pltpu
from jax.experimental.pallas import tpu_sc as plsc

B, K, NE, NH = 16384, 26, 32, 128
NC, NS, L = 2, 16, 16
NW = NC * NS
ROWS_W = B // NW
CHUNK = 4
NIDX = CHUNK * K
NSTEP = ROWS_W // CHUNK
NBUF = 4
NOUTER = NSTEP // NBUF


def _sc_gather_mul(cnt_table, val_table, goal_flat):
    mesh = plsc.VectorSubcoreMesh(core_axis_name="c", subcore_axis_name="s")

    @functools.partial(
        pl.kernel,
        mesh=mesh,
        out_type=jax.ShapeDtypeStruct((B, K * NE), jnp.float32),
        scratch_types=[
            pltpu.VMEM((ROWS_W * 2 * K,), jnp.int32),
            pltpu.VMEM((ROWS_W * K,), jnp.int32),
            pltpu.VMEM((ROWS_W * K,), jnp.int32),
            pltpu.VMEM((NBUF, NIDX, NE), jnp.float32),
            pltpu.VMEM((NBUF, NIDX, NE), jnp.float32),
            pltpu.VMEM((NBUF, CHUNK, K * NE), jnp.float32),
            [pltpu.SemaphoreType.DMA for _ in range(NBUF)],
            [pltpu.SemaphoreType.DMA for _ in range(NBUF)],
            [pltpu.SemaphoreType.DMA for _ in range(NBUF)],
        ],
        compiler_params=pltpu.CompilerParams(
            use_tc_tiling_on_sc=False, needs_layout_passes=False),
    )
    def k(cnt_hbm, val_hbm, goal_hbm, out_hbm,
          goal_v, cidx_v, vidx_v, crow_v, vrow_v, hrow_v, csems, vsems, osems):
        wid = lax.axis_index("s") * NC + lax.axis_index("c")
        wbase = wid * ROWS_W
        ibase = wid * (ROWS_W * K)
        pltpu.sync_copy(goal_hbm.at[pl.ds(2 * ibase, ROWS_W * 2 * K)], goal_v)

        def perm(i):
            return ((i & ~jnp.int32(511)) | ((i & 127) << 2)
                    | ((i >> 7) & 3))

        def deint(i, _):
            lane = lax.iota(jnp.int32, 16)
            t = i * 16
            cpos = 2 * (t + lane)
            cidx_v[pl.ds(t, 16)] = perm(plsc.load_gather(goal_v, [cpos]))
            vidx_v[pl.ds(t, 16)] = perm(plsc.load_gather(goal_v, [cpos + 1]))
            return 0

        lax.fori_loop(0, ROWS_W * K // 16, deint, 0)

        def start_gather(b, off):
            pltpu.async_copy(
                cnt_hbm.at[cidx_v.at[pl.ds(off, NIDX)]], crow_v.at[b],
                csems[b])
            pltpu.async_copy(
                val_hbm.at[vidx_v.at[pl.ds(off, NIDX)]], vrow_v.at[b],
                vsems[b])

        def wait_gather(b, off):
            pltpu.make_async_copy(
                cnt_hbm.at[cidx_v.at[pl.ds(off, NIDX)]], crow_v.at[b],
                csems[b]).wait()
            pltpu.make_async_copy(
                val_hbm.at[vidx_v.at[pl.ds(off, NIDX)]], vrow_v.at[b],
                vsems[b]).wait()

        for b in range(NBUF):
            start_gather(b, b * NIDX)

        def outer(o, _):
            for b in range(NBUF):
                off = (o * NBUF + b) * NIDX
                wait_gather(b, off)

                row0 = wbase + (o * NBUF + b) * CHUNK

                @pl.when(o >= 1)
                def _():
                    pltpu.make_async_copy(
                        hrow_v.at[b],
                        out_hbm.at[pl.ds(row0 - NBUF * CHUNK, CHUNK)],
                        osems[b]).wait()

                for c in range(CHUNK):
                    def mul(kk, _):
                        for e in range(0, NE, L):
                            hrow_v[b, c, pl.ds(kk * NE + e, L)] = (
                                crow_v[b, c * K + kk, pl.ds(e, L)]
                                * vrow_v[b, c * K + kk, pl.ds(e, L)])
                        return 0

                    lax.fori_loop(0, K, mul, 0, unroll=4)
                pltpu.async_copy(
                    hrow_v.at[b], out_hbm.at[pl.ds(row0, CHUNK)],
                    osems[b])

                @pl.when(o < NOUTER - 1)
                def _():
                    start_gather(b, off + NBUF * NIDX)
            return 0

        lax.fori_loop(0, NOUTER, outer, 0)
        for b in range(NBUF):
            row0 = wbase + (NSTEP - NBUF + b) * CHUNK
            pltpu.make_async_copy(
                hrow_v.at[b], out_hbm.at[pl.ds(row0, CHUNK)],
                osems[b]).wait()

    return k(cnt_table, val_table, goal_flat)


J4 = 32
TBN = 512 * J4


def _pack(x):
    z = x.reshape(32, J4, 4, 128)
    m = z.transpose(1, 2, 0, 3).reshape(J4, 128, 128)
    return jnp.swapaxes(m, 1, 2).reshape(TBN // 4, 128)


def _relayout_body(c_ref, v_ref, co_ref, vo_ref):
    co_ref[...] = _pack(c_ref[...])
    vo_ref[...] = _pack(v_ref[...])


def _tc_relayout(cnt_t, val_t):
    nrow = cnt_t.shape[1]
    g = (nrow + TBN - 1) // TBN
    return pl.pallas_call(
        _relayout_body,
        grid=(g,),
        in_specs=[
            pl.BlockSpec((32, TBN), lambda i: (0, i)),
            pl.BlockSpec((32, TBN), lambda i: (0, i)),
        ],
        out_specs=[
            pl.BlockSpec((TBN // 4, 128), lambda i: (i, 0)),
            pl.BlockSpec((TBN // 4, 128), lambda i: (i, 0)),
        ],
        out_shape=[
            jax.ShapeDtypeStruct((g * TBN // 4, 128), jnp.float32),
            jax.ShapeDtypeStruct((g * TBN // 4, 128), jnp.float32),
        ],
    )(cnt_t, val_t)


def _tc_body(h_ref, w_ref, b_ref, o_ref):
    h = jnp.tanh(h_ref[...])
    o_ref[...] = (
        jnp.dot(h, w_ref[...], preferred_element_type=jnp.float32)
        + b_ref[...])


def _tc_tanh_matmul(h, W, b):
    BM = 1024
    return pl.pallas_call(
        _tc_body,
        grid=(B // BM,),
        in_specs=[
            pl.BlockSpec((BM, K * NE), lambda i: (i, 0)),
            pl.BlockSpec((K * NE, NH), lambda i: (0, 0)),
            pl.BlockSpec((1, NH), lambda i: (0, 0)),
        ],
        out_specs=pl.BlockSpec((BM, NH), lambda i: (i, 0)),
        out_shape=jax.ShapeDtypeStruct((B, NH), jnp.float32),
    )(h, W, b)


@jax.jit
def kernel(goal, cnt_table, val_table, W, b):
    cnt_l, val_l = _tc_relayout(cnt_table.T, val_table.T)
    h = _sc_gather_mul(cnt_l.reshape(-1, NE), val_l.reshape(-1, NE),
                       goal.reshape(-1))
    return _tc_tanh_matmul(h, W, b.reshape(1, NH))
